# baseline (speedup 1.0000x reference)
import functools

import jax
import jax.numpy as jnp
from jax import lax
from jax.experimental import pallas as pl
from jax.experimental.pallas import tpu as pltpu
from jax.experimental.pallas import tpu_sc as plsc

T = 4096
D = 2048
H = 1024
E = 16
K = 2
C = 1024
SH = 1024
ROWS = E * C
TRASH = ROWS
BUF_ROWS = ROWS + 256

NW = 32
PAIRS = T * K
PPW = PAIRS // NW
TPW = T // NW

_bf16 = jnp.bfloat16
_f32 = jnp.float32
_i32 = jnp.int32


def _nt(a, b):
    return lax.dot_general(a, b, (((1,), (1,)), ((), ())),
                           preferred_element_type=_f32)


TB = 512


def _gate_body(x_ref, gw_ref, wts_ref, dstd_ref, dstc_ref, cnt_ref,
               carry_ref):
    g = pl.program_id(0)

    @pl.when(g == 0)
    def _():
        carry_ref[...] = jnp.zeros_like(carry_ref)

    xb = x_ref[...]
    gw = gw_ref[...]
    logits = _nt(xb.astype(_bf16), gw.astype(_bf16))
    m = jnp.max(logits, axis=1, keepdims=True)
    p = jnp.exp(logits - m)
    scores = p / jnp.sum(p, axis=1, keepdims=True)

    e_ids = lax.broadcasted_iota(_i32, (TB, E), 1)
    m1 = jnp.max(scores, axis=1, keepdims=True)
    i1 = jnp.min(jnp.where(scores == m1, e_ids, E), axis=1, keepdims=True)
    sc2 = jnp.where(e_ids == i1, -jnp.inf, scores)
    m2 = jnp.max(sc2, axis=1, keepdims=True)
    i2 = jnp.min(jnp.where(sc2 == m2, e_ids, E), axis=1, keepdims=True)

    a0 = (e_ids == i1).astype(_f32)
    a1 = (e_ids == i2).astype(_f32)
    s = a0 + a1
    lt = (lax.broadcasted_iota(_i32, (TB, TB), 0)
          > lax.broadcasted_iota(_i32, (TB, TB), 1)).astype(_bf16)
    pref = lax.dot_general(lt, s.astype(_bf16), (((1,), (0,)), ((), ())),
                           preferred_element_type=_f32)
    base = pref + carry_ref[...]
    pos0 = jnp.sum(jnp.where(a0 > 0, base, 0.0), axis=1, keepdims=True)
    pos1 = jnp.sum(jnp.where(a1 > 0, base, 0.0), axis=1, keepdims=True)
    carry_ref[...] = carry_ref[...] + jnp.sum(s, axis=0, keepdims=True)
    cnt_ref[...] = carry_ref[...].astype(_i32)

    keep0 = pos0 < C
    keep1 = pos1 < C
    slot0 = jnp.minimum(pos0, C - 1).astype(_i32)
    slot1 = jnp.minimum(pos1, C - 1).astype(_i32)
    d0c = i1 * C + slot0
    d1c = i2 * C + slot1
    r = lax.broadcasted_iota(_i32, (TB, 1), 0)
    trash = TRASH + (r % 256)
    dstd_ref[...] = jnp.concatenate(
        [jnp.where(keep0, d0c, trash), jnp.where(keep1, d1c, trash)], axis=1)
    dstc_ref[...] = jnp.concatenate([d0c, d1c], axis=1)
    wts_ref[...] = jnp.concatenate(
        [jnp.where(keep0, m1, 0.0), jnp.where(keep1, m2, 0.0)], axis=1)


def _k_gate(x, gate_w):
    return pl.pallas_call(
        _gate_body,
        grid=(T // TB,),
        in_specs=[
            pl.BlockSpec((TB, D), lambda g: (g, 0)),
            pl.BlockSpec((E, D), lambda g: (0, 0)),
        ],
        out_specs=[
            pl.BlockSpec((TB, K), lambda g: (g, 0)),
            pl.BlockSpec((TB, K), lambda g: (g, 0)),
            pl.BlockSpec((TB, K), lambda g: (g, 0)),
            pl.BlockSpec((1, E), lambda g: (0, 0)),
        ],
        out_shape=[
            jax.ShapeDtypeStruct((T, K), _f32),
            jax.ShapeDtypeStruct((T, K), _i32),
            jax.ShapeDtypeStruct((T, K), _i32),
            jax.ShapeDtypeStruct((1, E), _i32),
        ],
        scratch_shapes=[pltpu.VMEM((1, E), _f32)],
        compiler_params=pltpu.CompilerParams(
            dimension_semantics=("arbitrary",)),
    )(x, gate_w)


_CH = 16
_NCH = PPW // _CH


def _disp_body(x_hbm, dstd_hbm, buf_hbm, idx_v, rows0, rows1, gsem, ssem0,
               ssem1):
    wid = lax.axis_index("s") * 2 + lax.axis_index("c")
    base = wid * PPW
    pltpu.sync_copy(dstd_hbm.at[wid], idx_v)
    rows = (rows0, rows1)
    ssems = (ssem0, ssem1)
    pend = [None, None]
    iota = lax.broadcasted_iota(_i32, (_CH,), 0)
    for j in range(_NCH):
        b = j % 2
        if pend[b] is not None:
            pend[b].wait()
        iv = ((base + j * _CH + iota) >> 1).astype(_i32)
        pltpu.async_copy(x_hbm.at[iv], rows[b], gsem).wait()
        pend[b] = pltpu.async_copy(rows[b], buf_hbm.at[idx_v.at[j]], ssems[b])
    for b in range(2):
        if pend[b] is not None:
            pend[b].wait()


def _k_disp(x, dstd):
    mesh = plsc.VectorSubcoreMesh(core_axis_name="c", subcore_axis_name="s")
    return pl.kernel(
        _disp_body,
        out_type=jax.ShapeDtypeStruct((BUF_ROWS, D), _f32),
        mesh=mesh,
        scratch_types=[
            pltpu.VMEM((_NCH, _CH), _i32),
            pltpu.VMEM((_CH, D), _f32),
            pltpu.VMEM((_CH, D), _f32),
            pltpu.SemaphoreType.DMA,
            pltpu.SemaphoreType.DMA,
            pltpu.SemaphoreType.DMA,
        ],
    )(x, dstd)


BC = 128
NCB = C // BC


def _ffn_body(cnt_ref, buf_ref, w1_ref, w3_ref, w2_ref, ob_ref):
    e = pl.program_id(0)
    c = pl.program_id(1)
    cnt = jnp.minimum(cnt_ref[e], C)

    @pl.when(c * BC < cnt)
    def _():
        xb = buf_ref[...].astype(_bf16)
        hh = _nt(xb, w1_ref[0].astype(_bf16))
        uu = _nt(xb, w3_ref[0].astype(_bf16))
        act = (hh * lax.logistic(hh)) * uu
        ob_ref[...] = _nt(act.astype(_bf16), w2_ref[0].astype(_bf16))


def _k_ffn(cnt, buf, w1, w3, w2):
    grid_spec = pltpu.PrefetchScalarGridSpec(
        num_scalar_prefetch=1,
        grid=(E, NCB),
        in_specs=[
            pl.BlockSpec(
                (BC, D),
                lambda e, c, cnt: (
                    e * NCB + jnp.minimum(
                        c, jnp.maximum(
                            (jnp.minimum(cnt[e], C) + BC - 1) // BC - 1, 0)),
                    0)),
            pl.BlockSpec((1, H, D), lambda e, c, cnt: (e, 0, 0)),
            pl.BlockSpec((1, H, D), lambda e, c, cnt: (e, 0, 0)),
            pl.BlockSpec((1, D, H), lambda e, c, cnt: (e, 0, 0)),
        ],
        out_specs=pl.BlockSpec((BC, D), lambda e, c, cnt: (e * NCB + c, 0)),
    )
    return pl.pallas_call(
        _ffn_body,
        grid_spec=grid_spec,
        out_shape=jax.ShapeDtypeStruct((ROWS, D), _f32),
        compiler_params=pltpu.CompilerParams(
            dimension_semantics=("arbitrary", "arbitrary")),
    )(cnt, buf, w1, w3, w2)


STB = 256


def _shared_body(x_ref, sw1_ref, sw3_ref, sw2_ref, ysh_ref):
    xb = x_ref[...].astype(_bf16)
    hh = _nt(xb, sw1_ref[...].astype(_bf16))
    uu = _nt(xb, sw3_ref[...].astype(_bf16))
    act = (hh * lax.logistic(hh)) * uu
    ysh_ref[...] = _nt(act.astype(_bf16), sw2_ref[...].astype(_bf16))


def _k_shared(x, sw1, sw3, sw2):
    return pl.pallas_call(
        _shared_body,
        grid=(T // STB,),
        in_specs=[
            pl.BlockSpec((STB, D), lambda t: (t, 0)),
            pl.BlockSpec((H, D), lambda t: (0, 0)),
            pl.BlockSpec((H, D), lambda t: (0, 0)),
            pl.BlockSpec((D, H), lambda t: (0, 0)),
        ],
        out_specs=pl.BlockSpec((STB, D), lambda t: (t, 0)),
        out_shape=jax.ShapeDtypeStruct((T, D), _f32),
        compiler_params=pltpu.CompilerParams(
            dimension_semantics=("arbitrary",)),
    )(x, sw1, sw3, sw2)


ATB = 512


def _add_body(ysh_ref, a_ref, wts_ref, y_ref):
    a = a_ref[...]
    w0 = wts_ref[:, 0:1]
    w1 = wts_ref[:, 1:2]
    y_ref[...] = ysh_ref[...] + w0 * a[:, :D] + w1 * a[:, D:]


def _k_add(ysh, a2, wts):
    return pl.pallas_call(
        _add_body,
        grid=(T // ATB,),
        in_specs=[
            pl.BlockSpec((ATB, D), lambda t: (t, 0)),
            pl.BlockSpec((ATB, 2 * D), lambda t: (t, 0)),
            pl.BlockSpec((ATB, K), lambda t: (t, 0)),
        ],
        out_specs=pl.BlockSpec((ATB, D), lambda t: (t, 0)),
        out_shape=jax.ShapeDtypeStruct((T, D), _f32),
        compiler_params=pltpu.CompilerParams(
            dimension_semantics=("arbitrary",)),
    )(ysh, a2, wts)


def _comb_body(ob_hbm, dstc_hbm, a_hbm, idx_v, rows0, rows1, gsem, ssem0,
               ssem1):
    wid = lax.axis_index("s") * 2 + lax.axis_index("c")
    base = wid * PPW
    pltpu.sync_copy(dstc_hbm.at[wid], idx_v)
    rows = (rows0, rows1)
    ssems = (ssem0, ssem1)
    pend = [None, None]
    for j in range(_NCH):
        b = j % 2
        if pend[b] is not None:
            pend[b].wait()
        pltpu.async_copy(ob_hbm.at[idx_v.at[j]], rows[b], gsem).wait()
        pend[b] = pltpu.async_copy(
            rows[b], a_hbm.at[pl.ds(base + j * _CH, _CH)], ssems[b])
    for b in range(2):
        if pend[b] is not None:
            pend[b].wait()


def _k_comb(ob, dstc):
    mesh = plsc.VectorSubcoreMesh(core_axis_name="c", subcore_axis_name="s")
    return pl.kernel(
        _comb_body,
        out_type=jax.ShapeDtypeStruct((PAIRS, D), _f32),
        mesh=mesh,
        scratch_types=[
            pltpu.VMEM((_NCH, _CH), _i32),
            pltpu.VMEM((_CH, D), _f32),
            pltpu.VMEM((_CH, D), _f32),
            pltpu.SemaphoreType.DMA,
            pltpu.SemaphoreType.DMA,
            pltpu.SemaphoreType.DMA,
        ],
    )(ob, dstc)


def kernel(x, gate_w, w1, w3, w2, sw1, sw3, sw2):
    wts, dstd, dstc, cnt = _k_gate(x, gate_w)
    buf = _k_disp(x, dstd.reshape(NW, _NCH, _CH))
    ysh = _k_shared(x, sw1, sw3, sw2)
    ob = _k_ffn(cnt.reshape(E), buf, w1, w3, w2)
    a = _k_comb(ob, dstc.reshape(NW, _NCH, _CH))
    y = _k_add(ysh, a.reshape(T, 2 * D), wts)
    return y


# speedup vs baseline: 1.3007x; 1.3007x over previous
import functools

import jax
import jax.numpy as jnp
from jax import lax
from jax.experimental import pallas as pl
from jax.experimental.pallas import tpu as pltpu
from jax.experimental.pallas import tpu_sc as plsc

T = 4096
D = 2048
H = 1024
E = 16
K = 2
C = 1024
SH = 1024
ROWS = E * C
TRASH = ROWS
BUF_ROWS = ROWS + 256

NW = 32
PAIRS = T * K
PPW = PAIRS // NW
TPW = T // NW

_bf16 = jnp.bfloat16
_f32 = jnp.float32
_i32 = jnp.int32


def _nt(a, b):
    return lax.dot_general(a, b, (((1,), (1,)), ((), ())),
                           preferred_element_type=_f32)


TB = 512


def _gate_body(x_ref, gw_ref, wts_ref, dstd_ref, dstc_ref, cnt_ref,
               carry_ref):
    g = pl.program_id(0)

    @pl.when(g == 0)
    def _():
        carry_ref[...] = jnp.zeros_like(carry_ref)

    xb = x_ref[...]
    gw = gw_ref[...]
    logits = _nt(xb.astype(_bf16), gw.astype(_bf16))
    m = jnp.max(logits, axis=1, keepdims=True)
    p = jnp.exp(logits - m)
    scores = p / jnp.sum(p, axis=1, keepdims=True)

    e_ids = lax.broadcasted_iota(_i32, (TB, E), 1)
    m1 = jnp.max(scores, axis=1, keepdims=True)
    i1 = jnp.min(jnp.where(scores == m1, e_ids, E), axis=1, keepdims=True)
    sc2 = jnp.where(e_ids == i1, -jnp.inf, scores)
    m2 = jnp.max(sc2, axis=1, keepdims=True)
    i2 = jnp.min(jnp.where(sc2 == m2, e_ids, E), axis=1, keepdims=True)

    a0 = (e_ids == i1).astype(_f32)
    a1 = (e_ids == i2).astype(_f32)
    s = a0 + a1
    lt = (lax.broadcasted_iota(_i32, (TB, TB), 0)
          > lax.broadcasted_iota(_i32, (TB, TB), 1)).astype(_bf16)
    pref = lax.dot_general(lt, s.astype(_bf16), (((1,), (0,)), ((), ())),
                           preferred_element_type=_f32)
    base = pref + carry_ref[...]
    pos0 = jnp.sum(jnp.where(a0 > 0, base, 0.0), axis=1, keepdims=True)
    pos1 = jnp.sum(jnp.where(a1 > 0, base, 0.0), axis=1, keepdims=True)
    carry_ref[...] = carry_ref[...] + jnp.sum(s, axis=0, keepdims=True)
    cnt_ref[...] = carry_ref[...].astype(_i32)

    keep0 = pos0 < C
    keep1 = pos1 < C
    slot0 = jnp.minimum(pos0, C - 1).astype(_i32)
    slot1 = jnp.minimum(pos1, C - 1).astype(_i32)
    d0c = i1 * C + slot0
    d1c = i2 * C + slot1
    r = lax.broadcasted_iota(_i32, (TB, 1), 0)
    trash = TRASH + (r % 256)
    dstd_ref[...] = jnp.concatenate(
        [jnp.where(keep0, d0c, trash), jnp.where(keep1, d1c, trash)], axis=1)
    dstc_ref[...] = jnp.concatenate([d0c, d1c], axis=1)
    wts_ref[...] = jnp.concatenate(
        [jnp.where(keep0, m1, 0.0), jnp.where(keep1, m2, 0.0)], axis=1)


def _k_gate(x, gate_w):
    return pl.pallas_call(
        _gate_body,
        grid=(T // TB,),
        in_specs=[
            pl.BlockSpec((TB, D), lambda g: (g, 0)),
            pl.BlockSpec((E, D), lambda g: (0, 0)),
        ],
        out_specs=[
            pl.BlockSpec((TB, K), lambda g: (g, 0)),
            pl.BlockSpec((TB, K), lambda g: (g, 0)),
            pl.BlockSpec((TB, K), lambda g: (g, 0)),
            pl.BlockSpec((1, E), lambda g: (0, 0)),
        ],
        out_shape=[
            jax.ShapeDtypeStruct((T, K), _f32),
            jax.ShapeDtypeStruct((T, K), _i32),
            jax.ShapeDtypeStruct((T, K), _i32),
            jax.ShapeDtypeStruct((1, E), _i32),
        ],
        scratch_shapes=[pltpu.VMEM((1, E), _f32)],
        compiler_params=pltpu.CompilerParams(
            dimension_semantics=("arbitrary",)),
    )(x, gate_w)


_CH = 16
_NCH = PPW // _CH


def _disp_body(x_hbm, dstd_hbm, buf_hbm, idx_v, rows0, rows1, gsem, ssem0,
               ssem1):
    wid = lax.axis_index("s") * 2 + lax.axis_index("c")
    base = wid * PPW
    pltpu.sync_copy(dstd_hbm.at[wid], idx_v)
    rows = (rows0, rows1)
    ssems = (ssem0, ssem1)
    pend = [None, None]
    iota = lax.broadcasted_iota(_i32, (_CH,), 0)
    for j in range(_NCH):
        b = j % 2
        if pend[b] is not None:
            pend[b].wait()
        iv = ((base + j * _CH + iota) >> 1).astype(_i32)
        pltpu.async_copy(x_hbm.at[iv], rows[b], gsem).wait()
        pend[b] = pltpu.async_copy(rows[b], buf_hbm.at[idx_v.at[j]], ssems[b])
    for b in range(2):
        if pend[b] is not None:
            pend[b].wait()


def _k_disp(x, dstd):
    mesh = plsc.VectorSubcoreMesh(core_axis_name="c", subcore_axis_name="s")
    return pl.kernel(
        _disp_body,
        out_type=jax.ShapeDtypeStruct((BUF_ROWS, D), _f32),
        mesh=mesh,
        scratch_types=[
            pltpu.VMEM((_NCH, _CH), _i32),
            pltpu.VMEM((_CH, D), _f32),
            pltpu.VMEM((_CH, D), _f32),
            pltpu.SemaphoreType.DMA,
            pltpu.SemaphoreType.DMA,
            pltpu.SemaphoreType.DMA,
        ],
    )(x, dstd)


BC = 256
NCB = C // BC


def _ffn_body(cnt_ref, buf_ref, w1_ref, w3_ref, w2_ref, ob_ref):
    e = pl.program_id(0)
    c = pl.program_id(1)
    cnt = jnp.minimum(cnt_ref[e], C)

    @pl.when(c * BC < cnt)
    def _():
        xb = buf_ref[...].astype(_bf16)
        hh = _nt(xb, w1_ref[0].astype(_bf16))
        uu = _nt(xb, w3_ref[0].astype(_bf16))
        act = (hh * lax.logistic(hh)) * uu
        ob_ref[...] = _nt(act.astype(_bf16), w2_ref[0].astype(_bf16))


def _k_ffn(cnt, buf, w1, w3, w2):
    grid_spec = pltpu.PrefetchScalarGridSpec(
        num_scalar_prefetch=1,
        grid=(E, NCB),
        in_specs=[
            pl.BlockSpec(
                (BC, D),
                lambda e, c, cnt: (
                    e * NCB + jnp.minimum(
                        c, jnp.maximum(
                            (jnp.minimum(cnt[e], C) + BC - 1) // BC - 1, 0)),
                    0)),
            pl.BlockSpec((1, H, D), lambda e, c, cnt: (e, 0, 0)),
            pl.BlockSpec((1, H, D), lambda e, c, cnt: (e, 0, 0)),
            pl.BlockSpec((1, D, H), lambda e, c, cnt: (e, 0, 0)),
        ],
        out_specs=pl.BlockSpec((BC, D), lambda e, c, cnt: (e * NCB + c, 0)),
    )
    return pl.pallas_call(
        _ffn_body,
        grid_spec=grid_spec,
        out_shape=jax.ShapeDtypeStruct((ROWS, D), _f32),
        compiler_params=pltpu.CompilerParams(
            dimension_semantics=("arbitrary", "arbitrary")),
    )(cnt, buf, w1, w3, w2)


STB = 256


def _shared_body(x_ref, sw1_ref, sw3_ref, sw2_ref, ysh_ref):
    xb = x_ref[...].astype(_bf16)
    hh = _nt(xb, sw1_ref[...].astype(_bf16))
    uu = _nt(xb, sw3_ref[...].astype(_bf16))
    act = (hh * lax.logistic(hh)) * uu
    ysh_ref[...] = _nt(act.astype(_bf16), sw2_ref[...].astype(_bf16))


def _k_shared(x, sw1, sw3, sw2):
    return pl.pallas_call(
        _shared_body,
        grid=(T // STB,),
        in_specs=[
            pl.BlockSpec((STB, D), lambda t: (t, 0)),
            pl.BlockSpec((H, D), lambda t: (0, 0)),
            pl.BlockSpec((H, D), lambda t: (0, 0)),
            pl.BlockSpec((D, H), lambda t: (0, 0)),
        ],
        out_specs=pl.BlockSpec((STB, D), lambda t: (t, 0)),
        out_shape=jax.ShapeDtypeStruct((T, D), _f32),
        compiler_params=pltpu.CompilerParams(
            dimension_semantics=("arbitrary",)),
    )(x, sw1, sw3, sw2)


ATB = 512


def _add_body(ysh_ref, a_ref, wts_ref, y_ref):
    a = a_ref[...]
    w0 = wts_ref[:, 0:1]
    w1 = wts_ref[:, 1:2]
    y_ref[...] = ysh_ref[...] + w0 * a[:, :D] + w1 * a[:, D:]


def _k_add(ysh, a2, wts):
    return pl.pallas_call(
        _add_body,
        grid=(T // ATB,),
        in_specs=[
            pl.BlockSpec((ATB, D), lambda t: (t, 0)),
            pl.BlockSpec((ATB, 2 * D), lambda t: (t, 0)),
            pl.BlockSpec((ATB, K), lambda t: (t, 0)),
        ],
        out_specs=pl.BlockSpec((ATB, D), lambda t: (t, 0)),
        out_shape=jax.ShapeDtypeStruct((T, D), _f32),
        compiler_params=pltpu.CompilerParams(
            dimension_semantics=("arbitrary",)),
    )(ysh, a2, wts)


def _comb_body(ob_hbm, dstc_hbm, a_hbm, idx_v, rows0, rows1, gsem, ssem0,
               ssem1):
    wid = lax.axis_index("s") * 2 + lax.axis_index("c")
    base = wid * PPW
    pltpu.sync_copy(dstc_hbm.at[wid], idx_v)
    rows = (rows0, rows1)
    ssems = (ssem0, ssem1)
    pend = [None, None]
    for j in range(_NCH):
        b = j % 2
        if pend[b] is not None:
            pend[b].wait()
        pltpu.async_copy(ob_hbm.at[idx_v.at[j]], rows[b], gsem).wait()
        pend[b] = pltpu.async_copy(
            rows[b], a_hbm.at[pl.ds(base + j * _CH, _CH)], ssems[b])
    for b in range(2):
        if pend[b] is not None:
            pend[b].wait()


def _k_comb(ob, dstc):
    mesh = plsc.VectorSubcoreMesh(core_axis_name="c", subcore_axis_name="s")
    return pl.kernel(
        _comb_body,
        out_type=jax.ShapeDtypeStruct((PAIRS, D), _f32),
        mesh=mesh,
        scratch_types=[
            pltpu.VMEM((_NCH, _CH), _i32),
            pltpu.VMEM((_CH, D), _f32),
            pltpu.VMEM((_CH, D), _f32),
            pltpu.SemaphoreType.DMA,
            pltpu.SemaphoreType.DMA,
            pltpu.SemaphoreType.DMA,
        ],
    )(ob, dstc)


def kernel(x, gate_w, w1, w3, w2, sw1, sw3, sw2):
    wts, dstd, dstc, cnt = _k_gate(x, gate_w)
    buf = _k_disp(x, dstd.reshape(NW, _NCH, _CH))
    ysh = _k_shared(x, sw1, sw3, sw2)
    ob = _k_ffn(cnt.reshape(E), buf, w1, w3, w2)
    a = _k_comb(ob, dstc.reshape(NW, _NCH, _CH))
    y = _k_add(ysh, a.reshape(T, 2 * D), wts)
    return y


# speedup vs baseline: 1.3013x; 1.0004x over previous
import jax
import jax.numpy as jnp
from jax import lax
from jax.experimental import pallas as pl
from jax.experimental.pallas import tpu as pltpu
from jax.experimental.pallas import tpu_sc as plsc

T = 4096
D = 2048
H = 1024
E = 16
K = 2
C = 1024
SH = 1024
ROWS = E * C
TRASH = ROWS
BUF_ROWS = ROWS + 256

NW = 32
PAIRS = T * K
PPW = PAIRS // NW
TPW = T // NW

_bf16 = jnp.bfloat16
_f32 = jnp.float32
_i32 = jnp.int32


def _nt(a, b):
    return lax.dot_general(a, b, (((1,), (1,)), ((), ())),
                           preferred_element_type=_f32)


TB = 512


def _gate_body(x_ref, gw_ref, wts_ref, dstd_ref, dstc_ref, cnt_ref,
               carry_ref):
    g = pl.program_id(0)

    @pl.when(g == 0)
    def _():
        carry_ref[...] = jnp.zeros_like(carry_ref)

    xb = x_ref[...]
    gw = gw_ref[...]
    logits = _nt(xb.astype(_bf16), gw.astype(_bf16))
    m = jnp.max(logits, axis=1, keepdims=True)
    p = jnp.exp(logits - m)
    scores = p / jnp.sum(p, axis=1, keepdims=True)

    e_ids = lax.broadcasted_iota(_i32, (TB, E), 1)
    m1 = jnp.max(scores, axis=1, keepdims=True)
    i1 = jnp.min(jnp.where(scores == m1, e_ids, E), axis=1, keepdims=True)
    sc2 = jnp.where(e_ids == i1, -jnp.inf, scores)
    m2 = jnp.max(sc2, axis=1, keepdims=True)
    i2 = jnp.min(jnp.where(sc2 == m2, e_ids, E), axis=1, keepdims=True)

    a0 = (e_ids == i1).astype(_f32)
    a1 = (e_ids == i2).astype(_f32)
    s = a0 + a1
    lt = (lax.broadcasted_iota(_i32, (TB, TB), 0)
          > lax.broadcasted_iota(_i32, (TB, TB), 1)).astype(_bf16)
    pref = lax.dot_general(lt, s.astype(_bf16), (((1,), (0,)), ((), ())),
                           preferred_element_type=_f32)
    base = pref + carry_ref[...]
    pos0 = jnp.sum(jnp.where(a0 > 0, base, 0.0), axis=1, keepdims=True)
    pos1 = jnp.sum(jnp.where(a1 > 0, base, 0.0), axis=1, keepdims=True)
    carry_ref[...] = carry_ref[...] + jnp.sum(s, axis=0, keepdims=True)
    cnt_ref[...] = carry_ref[...].astype(_i32)

    keep0 = pos0 < C
    keep1 = pos1 < C
    slot0 = jnp.minimum(pos0, C - 1).astype(_i32)
    slot1 = jnp.minimum(pos1, C - 1).astype(_i32)
    d0c = i1 * C + slot0
    d1c = i2 * C + slot1
    r = lax.broadcasted_iota(_i32, (TB, 1), 0)
    trash = TRASH + (r % 256)
    dstd_ref[...] = jnp.concatenate(
        [jnp.where(keep0, d0c, trash), jnp.where(keep1, d1c, trash)], axis=1)
    dstc_ref[...] = jnp.concatenate([d0c, d1c], axis=1)
    wts_ref[...] = jnp.concatenate(
        [jnp.where(keep0, m1, 0.0), jnp.where(keep1, m2, 0.0)], axis=1)


def _k_gate(x, gate_w):
    return pl.pallas_call(
        _gate_body,
        grid=(T // TB,),
        in_specs=[
            pl.BlockSpec((TB, D), lambda g: (g, 0)),
            pl.BlockSpec((E, D), lambda g: (0, 0)),
        ],
        out_specs=[
            pl.BlockSpec((TB, K), lambda g: (g, 0)),
            pl.BlockSpec((TB, K), lambda g: (g, 0)),
            pl.BlockSpec((TB, K), lambda g: (g, 0)),
            pl.BlockSpec((1, E), lambda g: (0, 0)),
        ],
        out_shape=[
            jax.ShapeDtypeStruct((T, K), _f32),
            jax.ShapeDtypeStruct((T, K), _i32),
            jax.ShapeDtypeStruct((T, K), _i32),
            jax.ShapeDtypeStruct((1, E), _i32),
        ],
        scratch_shapes=[pltpu.VMEM((1, E), _f32)],
        compiler_params=pltpu.CompilerParams(
            dimension_semantics=("arbitrary",)),
    )(x, gate_w)


_CH = 16
_NCH = PPW // _CH


def _disp_body(x_hbm, dstd_hbm, buf_hbm, idx_v, rows0, rows1, gsem, ssem0,
               ssem1):
    wid = lax.axis_index("s") * 2 + lax.axis_index("c")
    base = wid * PPW
    pltpu.sync_copy(dstd_hbm.at[wid], idx_v)
    rows = (rows0, rows1)
    ssems = (ssem0, ssem1)
    pend = [None, None]
    iota = lax.broadcasted_iota(_i32, (_CH,), 0)
    for j in range(_NCH):
        b = j % 2
        if pend[b] is not None:
            pend[b].wait()
        iv = ((base + j * _CH + iota) >> 1).astype(_i32)
        pltpu.async_copy(x_hbm.at[iv], rows[b], gsem).wait()
        pend[b] = pltpu.async_copy(rows[b], buf_hbm.at[idx_v.at[j]], ssems[b])
    for b in range(2):
        if pend[b] is not None:
            pend[b].wait()


def _k_disp(x, dstd):
    mesh = plsc.VectorSubcoreMesh(core_axis_name="c", subcore_axis_name="s")
    return pl.kernel(
        _disp_body,
        out_type=jax.ShapeDtypeStruct((BUF_ROWS, D), _f32),
        mesh=mesh,
        scratch_types=[
            pltpu.VMEM((_NCH, _CH), _i32),
            pltpu.VMEM((_CH, D), _f32),
            pltpu.VMEM((_CH, D), _f32),
            pltpu.SemaphoreType.DMA,
            pltpu.SemaphoreType.DMA,
            pltpu.SemaphoreType.DMA,
        ],
    )(x, dstd)


BC = 256
NCB = C // BC


def _ffn_body(cnt_ref, buf_ref, w1_ref, w3_ref, w2_ref, ob_ref):
    e = pl.program_id(0)
    c = pl.program_id(1)
    cnt = jnp.minimum(cnt_ref[e], C)

    @pl.when(c * BC < cnt)
    def _():
        xb = buf_ref[...].astype(_bf16)
        hh = _nt(xb, w1_ref[0].astype(_bf16))
        uu = _nt(xb, w3_ref[0].astype(_bf16))
        act = (hh * lax.logistic(hh)) * uu
        ob_ref[...] = _nt(act.astype(_bf16), w2_ref[0].astype(_bf16))


def _k_ffn(cnt, buf, w1, w3, w2):
    grid_spec = pltpu.PrefetchScalarGridSpec(
        num_scalar_prefetch=1,
        grid=(E, NCB),
        in_specs=[
            pl.BlockSpec(
                (BC, D),
                lambda e, c, cnt: (
                    e * NCB + jnp.minimum(
                        c, jnp.maximum(
                            (jnp.minimum(cnt[e], C) + BC - 1) // BC - 1, 0)),
                    0)),
            pl.BlockSpec((1, H, D), lambda e, c, cnt: (e, 0, 0)),
            pl.BlockSpec((1, H, D), lambda e, c, cnt: (e, 0, 0)),
            pl.BlockSpec((1, D, H), lambda e, c, cnt: (e, 0, 0)),
        ],
        out_specs=pl.BlockSpec((BC, D), lambda e, c, cnt: (e * NCB + c, 0)),
    )
    return pl.pallas_call(
        _ffn_body,
        grid_spec=grid_spec,
        out_shape=jax.ShapeDtypeStruct((ROWS, D), _f32),
        compiler_params=pltpu.CompilerParams(
            dimension_semantics=("arbitrary", "arbitrary")),
    )(cnt, buf, w1, w3, w2)


STB = 256


def _shared_body(x_ref, sw1_ref, sw3_ref, sw2_ref, ysh_ref):
    xb = x_ref[...].astype(_bf16)
    hh = _nt(xb, sw1_ref[...].astype(_bf16))
    uu = _nt(xb, sw3_ref[...].astype(_bf16))
    act = (hh * lax.logistic(hh)) * uu
    ysh_ref[...] = _nt(act.astype(_bf16), sw2_ref[...].astype(_bf16))


def _k_shared(x, sw1, sw3, sw2):
    return pl.pallas_call(
        _shared_body,
        grid=(T // STB,),
        in_specs=[
            pl.BlockSpec((STB, D), lambda t: (t, 0)),
            pl.BlockSpec((H, D), lambda t: (0, 0)),
            pl.BlockSpec((H, D), lambda t: (0, 0)),
            pl.BlockSpec((D, H), lambda t: (0, 0)),
        ],
        out_specs=pl.BlockSpec((STB, D), lambda t: (t, 0)),
        out_shape=jax.ShapeDtypeStruct((T, D), _f32),
        compiler_params=pltpu.CompilerParams(
            dimension_semantics=("arbitrary",)),
    )(x, sw1, sw3, sw2)


ATB = 512


def _add_body(ysh_ref, a_ref, wts_ref, y_ref):
    a = a_ref[...]
    w0 = wts_ref[:, 0:1]
    w1 = wts_ref[:, 1:2]
    y_ref[...] = ysh_ref[...] + w0 * a[:, :D] + w1 * a[:, D:]


def _k_add(ysh, a2, wts):
    return pl.pallas_call(
        _add_body,
        grid=(T // ATB,),
        in_specs=[
            pl.BlockSpec((ATB, D), lambda t: (t, 0)),
            pl.BlockSpec((ATB, 2 * D), lambda t: (t, 0)),
            pl.BlockSpec((ATB, K), lambda t: (t, 0)),
        ],
        out_specs=pl.BlockSpec((ATB, D), lambda t: (t, 0)),
        out_shape=jax.ShapeDtypeStruct((T, D), _f32),
        compiler_params=pltpu.CompilerParams(
            dimension_semantics=("arbitrary",)),
    )(ysh, a2, wts)


def _comb_body(ob_hbm, dstc_hbm, a_hbm, idx_v, rows0, rows1, gsem, ssem0,
               ssem1):
    wid = lax.axis_index("s") * 2 + lax.axis_index("c")
    base = wid * PPW
    pltpu.sync_copy(dstc_hbm.at[wid], idx_v)
    rows = (rows0, rows1)
    ssems = (ssem0, ssem1)
    pend = [None, None]
    for j in range(_NCH):
        b = j % 2
        if pend[b] is not None:
            pend[b].wait()
        pltpu.async_copy(ob_hbm.at[idx_v.at[j]], rows[b], gsem).wait()
        pend[b] = pltpu.async_copy(
            rows[b], a_hbm.at[pl.ds(base + j * _CH, _CH)], ssems[b])
    for b in range(2):
        if pend[b] is not None:
            pend[b].wait()


def _k_comb(ob, dstc):
    mesh = plsc.VectorSubcoreMesh(core_axis_name="c", subcore_axis_name="s")
    return pl.kernel(
        _comb_body,
        out_type=jax.ShapeDtypeStruct((PAIRS, D), _f32),
        mesh=mesh,
        scratch_types=[
            pltpu.VMEM((_NCH, _CH), _i32),
            pltpu.VMEM((_CH, D), _f32),
            pltpu.VMEM((_CH, D), _f32),
            pltpu.SemaphoreType.DMA,
            pltpu.SemaphoreType.DMA,
            pltpu.SemaphoreType.DMA,
        ],
    )(ob, dstc)


def kernel(x, gate_w, w1, w3, w2, sw1, sw3, sw2):
    wts, dstd, dstc, cnt = _k_gate(x, gate_w)
    buf = _k_disp(x, dstd.reshape(NW, _NCH, _CH))
    ysh = _k_shared(x, sw1, sw3, sw2)
    ob = _k_ffn(cnt.reshape(E), buf, w1, w3, w2)
    a = _k_comb(ob, dstc.reshape(NW, _NCH, _CH))
    y = _k_add(ysh, a.reshape(T, 2 * D), wts)
    return y
